# Initial kernel scaffold; baseline (speedup 1.0000x reference)
import jax
import jax.numpy as jnp
from jax.experimental import pallas as pl


def kernel(z, embedding, batch_size, n_train):
    raise NotImplementedError("write your pallas kernel here")


# speedup vs baseline: 5.6876x; 5.6876x over previous
import functools

import jax
import jax.numpy as jnp
from jax import lax
from jax.experimental import pallas as pl
from jax.experimental.pallas import tpu as pltpu
from jax.experimental.pallas import tpu_sc as plsc

N_E = 8192
E_DIM = 256
BETA = 0.25
TOK_BLK = 128


def _vq_tc_body(z_ref, emb_ref, zs_ref, es_ref,
                oh_ref, idx_ref, cnt_ref, quant_ref, commit_ref, perp_ref,
                ssq_ref):
    i = pl.program_id(0)
    nsteps = pl.num_programs(0)
    n_tok = nsteps * TOK_BLK
    n_elem = n_tok * E_DIM

    z_blk = z_ref[...]
    emb = emb_ref[...]
    m = lax.dot_general(z_blk, emb, (((1,), (1,)), ((), ())),
                        preferred_element_type=jnp.float32)
    d = (zs_ref[...] + es_ref[...]) - 2.0 * m

    vmin = jnp.min(d, axis=1, keepdims=True)
    iota = lax.broadcasted_iota(jnp.int32, (TOK_BLK, N_E), 1)
    idx = jnp.min(jnp.where(d == vmin, iota, N_E), axis=1)
    onehot = (iota == idx[:, None]).astype(jnp.float32)

    oh_ref[...] = onehot
    idx_ref[...] = idx.reshape(1, 1, TOK_BLK)

    @pl.when(i == 0)
    def _init():
        cnt_ref[...] = jnp.zeros_like(cnt_ref)
        ssq_ref[0, 0] = 0.0

    cnt_ref[...] += jnp.sum(onehot, axis=0, keepdims=True)
    ssq_ref[0, 0] += jnp.sum(vmin)

    @pl.when(i == nsteps - 1)
    def _finalize():
        total = ssq_ref[0, 0]
        quant_ref[0, 0] = total / n_elem
        commit_ref[0, 0] = BETA * (total / n_elem)
        e_mean = cnt_ref[...] * (1.0 / n_tok)
        ent = jnp.sum(e_mean * jnp.log(e_mean + 1e-10))
        perp_ref[0, 0] = jnp.exp(-ent)


def _vq_tc(z2, emb, zs, es, interpret=False):
    n_tok = z2.shape[0]
    nblk = n_tok // TOK_BLK
    out_shapes = (
        jax.ShapeDtypeStruct((n_tok, N_E), jnp.float32),
        jax.ShapeDtypeStruct((nblk, 1, TOK_BLK), jnp.int32),
        jax.ShapeDtypeStruct((1, N_E), jnp.float32),
        jax.ShapeDtypeStruct((1, 1), jnp.float32),
        jax.ShapeDtypeStruct((1, 1), jnp.float32),
        jax.ShapeDtypeStruct((1, 1), jnp.float32),
    )
    grid = (nblk,)
    return pl.pallas_call(
        _vq_tc_body,
        grid=grid,
        in_specs=[
            pl.BlockSpec((TOK_BLK, E_DIM), lambda i: (i, 0)),
            pl.BlockSpec((N_E, E_DIM), lambda i: (0, 0)),
            pl.BlockSpec((TOK_BLK, 1), lambda i: (i, 0)),
            pl.BlockSpec((1, N_E), lambda i: (0, 0)),
        ],
        out_specs=(
            pl.BlockSpec((TOK_BLK, N_E), lambda i: (i, 0)),
            pl.BlockSpec((1, 1, TOK_BLK), lambda i: (i, 0, 0)),
            pl.BlockSpec((1, N_E), lambda i: (0, 0)),
            pl.BlockSpec((1, 1), lambda i: (0, 0), memory_space=pltpu.SMEM),
            pl.BlockSpec((1, 1), lambda i: (0, 0), memory_space=pltpu.SMEM),
            pl.BlockSpec((1, 1), lambda i: (0, 0), memory_space=pltpu.SMEM),
        ),
        out_shape=out_shapes,
        scratch_shapes=[pltpu.SMEM((1, 1), jnp.float32)],
        interpret=interpret,
    )(z2, emb, zs, es)


def _sc_gather(emb, idx_flat):
    info = plsc.get_sparse_core_info()
    nw = info.num_cores * info.num_subcores
    n_tok = idx_flat.shape[0]
    b_per_w = n_tok // nw
    half = b_per_w // 2
    mesh = plsc.VectorSubcoreMesh(core_axis_name="c", subcore_axis_name="s")

    @functools.partial(
        pl.kernel, mesh=mesh,
        out_type=jax.ShapeDtypeStruct((n_tok, E_DIM), jnp.float32),
        scratch_types=[
            pltpu.VMEM((half,), jnp.int32),
            pltpu.VMEM((half,), jnp.int32),
            pltpu.VMEM((b_per_w, E_DIM), jnp.float32),
            pltpu.SemaphoreType.DMA,
        ],
    )
    def k(emb_hbm, idx_hbm, out_hbm, idx_v0, idx_v1, rows_v, sem):
        wid = lax.axis_index("s") * info.num_cores + lax.axis_index("c")
        base = wid * b_per_w
        pltpu.sync_copy(idx_hbm.at[pl.ds(base, half)], idx_v0)
        pltpu.sync_copy(idx_hbm.at[pl.ds(base + half, half)], idx_v1)
        c0 = pltpu.async_copy(emb_hbm.at[idx_v0],
                              rows_v.at[pl.ds(0, half)], sem)
        c1 = pltpu.async_copy(emb_hbm.at[idx_v1],
                              rows_v.at[pl.ds(half, half)], sem)
        c0.wait()
        c1.wait()
        pltpu.sync_copy(rows_v, out_hbm.at[pl.ds(base, b_per_w)])

    return k(emb, idx_flat)


def kernel(z, embedding, batch_size, n_train):
    z2 = z.reshape(-1, E_DIM)
    zs = jnp.sum(z2 ** 2, axis=1, keepdims=True)
    es = jnp.sum(embedding ** 2, axis=1)

    onehot, idx3, _cnt, quant, commit, perp = _vq_tc(
        z2, embedding, zs, es.reshape(1, N_E))
    idx_flat = idx3.reshape(-1)
    z_q = _sc_gather(embedding, idx_flat)

    return (quant.reshape(()), commit.reshape(()), z_q.reshape(z.shape),
            perp.reshape(()), onehot, idx_flat[:, None].astype(jnp.int32))


# speedup vs baseline: 7.4083x; 1.3025x over previous
import functools

import jax
import jax.numpy as jnp
from jax import lax
from jax.experimental import pallas as pl
from jax.experimental.pallas import tpu as pltpu
from jax.experimental.pallas import tpu_sc as plsc

N_E = 8192
E_DIM = 256
BETA = 0.25
TOK_BLK = 256


def _vq_tc_body(z_ref, emb_ref, zs_ref, es_ref,
                oh_ref, idx_ref, cnt_ref, quant_ref, commit_ref, perp_ref,
                ssq_ref):
    i = pl.program_id(0)
    nsteps = pl.num_programs(0)
    n_tok = nsteps * TOK_BLK
    n_elem = n_tok * E_DIM

    z_blk = z_ref[...]
    emb = emb_ref[...]
    m = lax.dot_general(z_blk, emb, (((1,), (1,)), ((), ())),
                        preferred_element_type=jnp.float32)
    d = (zs_ref[...] + es_ref[...]) - 2.0 * m

    vmin = jnp.min(d, axis=1, keepdims=True)
    iota = lax.broadcasted_iota(jnp.int32, (TOK_BLK, N_E), 1)
    idx = jnp.min(jnp.where(d == vmin, iota, N_E), axis=1)
    onehot = (iota == idx[:, None]).astype(jnp.float32)

    oh_ref[...] = onehot
    idx_ref[...] = idx.reshape(1, 1, TOK_BLK)

    @pl.when(i == 0)
    def _init():
        cnt_ref[...] = jnp.zeros_like(cnt_ref)
        ssq_ref[0, 0] = 0.0

    cnt_ref[...] += jnp.sum(onehot, axis=0, keepdims=True)
    ssq_ref[0, 0] += jnp.sum(vmin)

    @pl.when(i == nsteps - 1)
    def _finalize():
        total = ssq_ref[0, 0]
        quant_ref[0, 0] = total / n_elem
        commit_ref[0, 0] = BETA * (total / n_elem)
        e_mean = cnt_ref[...] * (1.0 / n_tok)
        ent = jnp.sum(e_mean * jnp.log(e_mean + 1e-10))
        perp_ref[0, 0] = jnp.exp(-ent)


def _vq_tc(z2, emb, zs, es, interpret=False):
    n_tok = z2.shape[0]
    nblk = n_tok // TOK_BLK
    out_shapes = (
        jax.ShapeDtypeStruct((n_tok, N_E), jnp.float32),
        jax.ShapeDtypeStruct((nblk, 1, TOK_BLK), jnp.int32),
        jax.ShapeDtypeStruct((1, N_E), jnp.float32),
        jax.ShapeDtypeStruct((1, 1), jnp.float32),
        jax.ShapeDtypeStruct((1, 1), jnp.float32),
        jax.ShapeDtypeStruct((1, 1), jnp.float32),
    )
    grid = (nblk,)
    return pl.pallas_call(
        _vq_tc_body,
        grid=grid,
        in_specs=[
            pl.BlockSpec((TOK_BLK, E_DIM), lambda i: (i, 0)),
            pl.BlockSpec((N_E, E_DIM), lambda i: (0, 0)),
            pl.BlockSpec((TOK_BLK, 1), lambda i: (i, 0)),
            pl.BlockSpec((1, N_E), lambda i: (0, 0)),
        ],
        out_specs=(
            pl.BlockSpec((TOK_BLK, N_E), lambda i: (i, 0)),
            pl.BlockSpec((1, 1, TOK_BLK), lambda i: (i, 0, 0)),
            pl.BlockSpec((1, N_E), lambda i: (0, 0)),
            pl.BlockSpec((1, 1), lambda i: (0, 0), memory_space=pltpu.SMEM),
            pl.BlockSpec((1, 1), lambda i: (0, 0), memory_space=pltpu.SMEM),
            pl.BlockSpec((1, 1), lambda i: (0, 0), memory_space=pltpu.SMEM),
        ),
        out_shape=out_shapes,
        scratch_shapes=[pltpu.SMEM((1, 1), jnp.float32)],
        interpret=interpret,
    )(z2, emb, zs, es)


def _sc_gather(emb, idx_flat):
    info = plsc.get_sparse_core_info()
    nw = info.num_cores * info.num_subcores
    n_tok = idx_flat.shape[0]
    b_per_w = n_tok // nw
    half = b_per_w // 2
    mesh = plsc.VectorSubcoreMesh(core_axis_name="c", subcore_axis_name="s")

    @functools.partial(
        pl.kernel, mesh=mesh,
        out_type=jax.ShapeDtypeStruct((n_tok, E_DIM), jnp.float32),
        scratch_types=[
            pltpu.VMEM((half,), jnp.int32),
            pltpu.VMEM((half,), jnp.int32),
            pltpu.VMEM((b_per_w, E_DIM), jnp.float32),
            pltpu.SemaphoreType.DMA,
        ],
    )
    def k(emb_hbm, idx_hbm, out_hbm, idx_v0, idx_v1, rows_v, sem):
        wid = lax.axis_index("s") * info.num_cores + lax.axis_index("c")
        base = wid * b_per_w
        pltpu.sync_copy(idx_hbm.at[pl.ds(base, half)], idx_v0)
        pltpu.sync_copy(idx_hbm.at[pl.ds(base + half, half)], idx_v1)
        c0 = pltpu.async_copy(emb_hbm.at[idx_v0],
                              rows_v.at[pl.ds(0, half)], sem)
        c1 = pltpu.async_copy(emb_hbm.at[idx_v1],
                              rows_v.at[pl.ds(half, half)], sem)
        c0.wait()
        c1.wait()
        pltpu.sync_copy(rows_v, out_hbm.at[pl.ds(base, b_per_w)])

    return k(emb, idx_flat)


def kernel(z, embedding, batch_size, n_train):
    z2 = z.reshape(-1, E_DIM)
    zs = jnp.sum(z2 ** 2, axis=1, keepdims=True)
    es = jnp.sum(embedding ** 2, axis=1)

    onehot, idx3, _cnt, quant, commit, perp = _vq_tc(
        z2, embedding, zs, es.reshape(1, N_E))
    idx_flat = idx3.reshape(-1)
    z_q = _sc_gather(embedding, idx_flat)

    return (quant.reshape(()), commit.reshape(()), z_q.reshape(z.shape),
            perp.reshape(()), onehot, idx_flat[:, None].astype(jnp.int32))


# speedup vs baseline: 7.9121x; 1.0680x over previous
import functools

import jax
import jax.numpy as jnp
from jax import lax
from jax.experimental import pallas as pl
from jax.experimental.pallas import tpu as pltpu
from jax.experimental.pallas import tpu_sc as plsc

N_E = 8192
E_DIM = 256
BETA = 0.25
TOK_BLK = 512


def _vq_tc_body(z_ref, emb_ref, zs_ref, es_ref,
                oh_ref, idx_ref, cnt_ref, quant_ref, commit_ref, perp_ref,
                ssq_ref):
    i = pl.program_id(0)
    nsteps = pl.num_programs(0)
    n_tok = nsteps * TOK_BLK
    n_elem = n_tok * E_DIM

    z_blk = z_ref[...]
    emb = emb_ref[...]
    m = lax.dot_general(z_blk, emb, (((1,), (1,)), ((), ())),
                        preferred_element_type=jnp.float32)
    d = (zs_ref[...] + es_ref[...]) - 2.0 * m

    vmin = jnp.min(d, axis=1, keepdims=True)
    iota = lax.broadcasted_iota(jnp.int32, (TOK_BLK, N_E), 1)
    idx = jnp.min(jnp.where(d == vmin, iota, N_E), axis=1)
    onehot = (iota == idx[:, None]).astype(jnp.float32)

    oh_ref[...] = onehot
    idx_ref[...] = idx.reshape(1, 1, TOK_BLK)

    @pl.when(i == 0)
    def _init():
        cnt_ref[...] = jnp.zeros_like(cnt_ref)
        ssq_ref[0, 0] = 0.0

    cnt_ref[...] += jnp.sum(onehot, axis=0, keepdims=True)
    ssq_ref[0, 0] += jnp.sum(vmin)

    @pl.when(i == nsteps - 1)
    def _finalize():
        total = ssq_ref[0, 0]
        quant_ref[0, 0] = total / n_elem
        commit_ref[0, 0] = BETA * (total / n_elem)
        e_mean = cnt_ref[...] * (1.0 / n_tok)
        ent = jnp.sum(e_mean * jnp.log(e_mean + 1e-10))
        perp_ref[0, 0] = jnp.exp(-ent)


def _vq_tc(z2, emb, zs, es, interpret=False):
    n_tok = z2.shape[0]
    nblk = n_tok // TOK_BLK
    out_shapes = (
        jax.ShapeDtypeStruct((n_tok, N_E), jnp.float32),
        jax.ShapeDtypeStruct((nblk, 1, TOK_BLK), jnp.int32),
        jax.ShapeDtypeStruct((1, N_E), jnp.float32),
        jax.ShapeDtypeStruct((1, 1), jnp.float32),
        jax.ShapeDtypeStruct((1, 1), jnp.float32),
        jax.ShapeDtypeStruct((1, 1), jnp.float32),
    )
    grid = (nblk,)
    return pl.pallas_call(
        _vq_tc_body,
        grid=grid,
        in_specs=[
            pl.BlockSpec((TOK_BLK, E_DIM), lambda i: (i, 0)),
            pl.BlockSpec((N_E, E_DIM), lambda i: (0, 0)),
            pl.BlockSpec((TOK_BLK, 1), lambda i: (i, 0)),
            pl.BlockSpec((1, N_E), lambda i: (0, 0)),
        ],
        out_specs=(
            pl.BlockSpec((TOK_BLK, N_E), lambda i: (i, 0)),
            pl.BlockSpec((1, 1, TOK_BLK), lambda i: (i, 0, 0)),
            pl.BlockSpec((1, N_E), lambda i: (0, 0)),
            pl.BlockSpec((1, 1), lambda i: (0, 0), memory_space=pltpu.SMEM),
            pl.BlockSpec((1, 1), lambda i: (0, 0), memory_space=pltpu.SMEM),
            pl.BlockSpec((1, 1), lambda i: (0, 0), memory_space=pltpu.SMEM),
        ),
        out_shape=out_shapes,
        scratch_shapes=[pltpu.SMEM((1, 1), jnp.float32)],
        interpret=interpret,
    )(z2, emb, zs, es)


def _sc_gather(emb, idx_flat):
    info = plsc.get_sparse_core_info()
    nw = info.num_cores * info.num_subcores
    n_tok = idx_flat.shape[0]
    b_per_w = n_tok // nw
    half = b_per_w // 2
    mesh = plsc.VectorSubcoreMesh(core_axis_name="c", subcore_axis_name="s")

    @functools.partial(
        pl.kernel, mesh=mesh,
        out_type=jax.ShapeDtypeStruct((n_tok, E_DIM), jnp.float32),
        scratch_types=[
            pltpu.VMEM((half,), jnp.int32),
            pltpu.VMEM((half,), jnp.int32),
            pltpu.VMEM((b_per_w, E_DIM), jnp.float32),
            pltpu.SemaphoreType.DMA,
        ],
    )
    def k(emb_hbm, idx_hbm, out_hbm, idx_v0, idx_v1, rows_v, sem):
        wid = lax.axis_index("s") * info.num_cores + lax.axis_index("c")
        base = wid * b_per_w
        pltpu.sync_copy(idx_hbm.at[pl.ds(base, half)], idx_v0)
        pltpu.sync_copy(idx_hbm.at[pl.ds(base + half, half)], idx_v1)
        c0 = pltpu.async_copy(emb_hbm.at[idx_v0],
                              rows_v.at[pl.ds(0, half)], sem)
        c1 = pltpu.async_copy(emb_hbm.at[idx_v1],
                              rows_v.at[pl.ds(half, half)], sem)
        c0.wait()
        c1.wait()
        pltpu.sync_copy(rows_v, out_hbm.at[pl.ds(base, b_per_w)])

    return k(emb, idx_flat)


def kernel(z, embedding, batch_size, n_train):
    z2 = z.reshape(-1, E_DIM)
    zs = jnp.sum(z2 ** 2, axis=1, keepdims=True)
    es = jnp.sum(embedding ** 2, axis=1)

    onehot, idx3, _cnt, quant, commit, perp = _vq_tc(
        z2, embedding, zs, es.reshape(1, N_E))
    idx_flat = idx3.reshape(-1)
    z_q = _sc_gather(embedding, idx_flat)

    return (quant.reshape(()), commit.reshape(()), z_q.reshape(z.shape),
            perp.reshape(()), onehot, idx_flat[:, None].astype(jnp.int32))


# speedup vs baseline: 8.6066x; 1.0878x over previous
import functools

import jax
import jax.numpy as jnp
from jax import lax
from jax.experimental import pallas as pl
from jax.experimental.pallas import tpu as pltpu
from jax.experimental.pallas import tpu_sc as plsc

N_E = 8192
E_DIM = 256
BETA = 0.25
TOK_BLK = 512


def _vq_tc_body(z_ref, emb_ref, zs_ref, es_ref,
                oh_ref, idx_ref, cnt_ref, quant_ref, commit_ref, perp_ref,
                ssq_ref):
    i = pl.program_id(0)
    nsteps = pl.num_programs(0)
    n_tok = nsteps * TOK_BLK
    n_elem = n_tok * E_DIM

    z_blk = z_ref[...]
    emb = emb_ref[...]
    m = lax.dot_general(z_blk, emb, (((1,), (1,)), ((), ())),
                        preferred_element_type=jnp.float32)
    zs = zs_ref[...]
    es = es_ref[...]

    LANES = 128
    nchunk = N_E // LANES
    run_min = jnp.full((TOK_BLK, LANES), jnp.inf, dtype=jnp.float32)
    run_col = jnp.zeros((TOK_BLK, LANES), dtype=jnp.float32)
    for k in range(nchunk):
        mk = m[:, k * LANES:(k + 1) * LANES]
        dk = (zs + es[:, k * LANES:(k + 1) * LANES]) - 2.0 * mk
        lt = dk < run_min
        run_min = jnp.where(lt, dk, run_min)
        run_col = jnp.where(lt, jnp.float32(k), run_col)

    lane = lax.broadcasted_iota(jnp.int32, (TOK_BLK, LANES), 1).astype(jnp.float32)
    run_idx = run_col * LANES + lane
    vmin = jnp.min(run_min, axis=1, keepdims=True)
    idxf = jnp.min(jnp.where(run_min == vmin, run_idx, jnp.float32(N_E)),
                   axis=1)
    idx = idxf.astype(jnp.int32)

    iota = lax.broadcasted_iota(jnp.int32, (TOK_BLK, N_E), 1).astype(jnp.float32)
    onehot = (iota == idxf[:, None]).astype(jnp.float32)

    oh_ref[...] = onehot
    idx_ref[...] = idx.reshape(1, 1, TOK_BLK)

    @pl.when(i == 0)
    def _init():
        cnt_ref[...] = jnp.zeros_like(cnt_ref)
        ssq_ref[0, 0] = 0.0

    ones_row = jnp.ones((1, TOK_BLK), dtype=jnp.float32)
    cnt_ref[...] += lax.dot_general(ones_row, onehot, (((1,), (0,)), ((), ())),
                                    preferred_element_type=jnp.float32)
    ssq_ref[0, 0] += jnp.sum(vmin)

    @pl.when(i == nsteps - 1)
    def _finalize():
        total = ssq_ref[0, 0]
        quant_ref[0, 0] = total / n_elem
        commit_ref[0, 0] = BETA * (total / n_elem)
        e_mean = cnt_ref[...] * (1.0 / n_tok)
        ent = jnp.sum(e_mean * jnp.log(e_mean + 1e-10))
        perp_ref[0, 0] = jnp.exp(-ent)


def _vq_tc(z2, emb, zs, es, interpret=False):
    n_tok = z2.shape[0]
    nblk = n_tok // TOK_BLK
    out_shapes = (
        jax.ShapeDtypeStruct((n_tok, N_E), jnp.float32),
        jax.ShapeDtypeStruct((nblk, 1, TOK_BLK), jnp.int32),
        jax.ShapeDtypeStruct((1, N_E), jnp.float32),
        jax.ShapeDtypeStruct((1, 1), jnp.float32),
        jax.ShapeDtypeStruct((1, 1), jnp.float32),
        jax.ShapeDtypeStruct((1, 1), jnp.float32),
    )
    grid = (nblk,)
    return pl.pallas_call(
        _vq_tc_body,
        grid=grid,
        in_specs=[
            pl.BlockSpec((TOK_BLK, E_DIM), lambda i: (i, 0)),
            pl.BlockSpec((N_E, E_DIM), lambda i: (0, 0)),
            pl.BlockSpec((TOK_BLK, 1), lambda i: (i, 0)),
            pl.BlockSpec((1, N_E), lambda i: (0, 0)),
        ],
        out_specs=(
            pl.BlockSpec((TOK_BLK, N_E), lambda i: (i, 0)),
            pl.BlockSpec((1, 1, TOK_BLK), lambda i: (i, 0, 0)),
            pl.BlockSpec((1, N_E), lambda i: (0, 0)),
            pl.BlockSpec((1, 1), lambda i: (0, 0), memory_space=pltpu.SMEM),
            pl.BlockSpec((1, 1), lambda i: (0, 0), memory_space=pltpu.SMEM),
            pl.BlockSpec((1, 1), lambda i: (0, 0), memory_space=pltpu.SMEM),
        ),
        out_shape=out_shapes,
        scratch_shapes=[pltpu.SMEM((1, 1), jnp.float32)],
        interpret=interpret,
    )(z2, emb, zs, es)


def _sc_gather(emb, idx_flat):
    info = plsc.get_sparse_core_info()
    nw = info.num_cores * info.num_subcores
    n_tok = idx_flat.shape[0]
    b_per_w = n_tok // nw
    half = b_per_w // 2
    mesh = plsc.VectorSubcoreMesh(core_axis_name="c", subcore_axis_name="s")

    @functools.partial(
        pl.kernel, mesh=mesh,
        out_type=jax.ShapeDtypeStruct((n_tok, E_DIM), jnp.float32),
        scratch_types=[
            pltpu.VMEM((half,), jnp.int32),
            pltpu.VMEM((half,), jnp.int32),
            pltpu.VMEM((b_per_w, E_DIM), jnp.float32),
            pltpu.SemaphoreType.DMA,
        ],
    )
    def k(emb_hbm, idx_hbm, out_hbm, idx_v0, idx_v1, rows_v, sem):
        wid = lax.axis_index("s") * info.num_cores + lax.axis_index("c")
        base = wid * b_per_w
        pltpu.sync_copy(idx_hbm.at[pl.ds(base, half)], idx_v0)
        pltpu.sync_copy(idx_hbm.at[pl.ds(base + half, half)], idx_v1)
        c0 = pltpu.async_copy(emb_hbm.at[idx_v0],
                              rows_v.at[pl.ds(0, half)], sem)
        c1 = pltpu.async_copy(emb_hbm.at[idx_v1],
                              rows_v.at[pl.ds(half, half)], sem)
        c0.wait()
        c1.wait()
        pltpu.sync_copy(rows_v, out_hbm.at[pl.ds(base, b_per_w)])

    return k(emb, idx_flat)


def kernel(z, embedding, batch_size, n_train):
    z2 = z.reshape(-1, E_DIM)
    zs = jnp.sum(z2 ** 2, axis=1, keepdims=True)
    es = jnp.sum(embedding ** 2, axis=1)

    onehot, idx3, _cnt, quant, commit, perp = _vq_tc(
        z2, embedding, zs, es.reshape(1, N_E))
    idx_flat = idx3.reshape(-1)
    z_q = _sc_gather(embedding, idx_flat)

    return (quant.reshape(()), commit.reshape(()), z_q.reshape(z.shape),
            perp.reshape(()), onehot, idx_flat[:, None].astype(jnp.int32))
